# Initial kernel scaffold; baseline (speedup 1.0000x reference)
import jax
import jax.numpy as jnp
from jax.experimental import pallas as pl


def kernel(location, embedding_matrix):
    raise NotImplementedError("write your pallas kernel here")


# speedup vs baseline: 4.7508x; 4.7508x over previous
import functools

import jax
import jax.numpy as jnp
from jax import lax
from jax.experimental import pallas as pl
from jax.experimental.pallas import tpu as pltpu
from jax.experimental.pallas import tpu_sc as plsc

_INFO = plsc.get_sparse_core_info()
_NC, _NS = _INFO.num_cores, _INFO.num_subcores
_NW = _NC * _NS

_IDX_W = 128
_ROWS_PER_IT = 4
_CHUNK = _ROWS_PER_IT * _IDX_W


def _make_sc_gather(n_rows128, dim):
    assert n_rows128 % (_NW * _ROWS_PER_IT) == 0
    rows_per_w = n_rows128 // _NW
    iters = rows_per_w // _ROWS_PER_IT
    n_total = n_rows128 * _IDX_W

    mesh = plsc.VectorSubcoreMesh(core_axis_name="c", subcore_axis_name="s")

    @functools.partial(
        pl.kernel,
        mesh=mesh,
        out_type=jax.ShapeDtypeStruct((n_total, dim), jnp.float32),
        scratch_types=[
            pltpu.VMEM((_ROWS_PER_IT, _IDX_W), jnp.int32),
            pltpu.VMEM((_CHUNK, dim), jnp.float32),
            pltpu.SemaphoreType.DMA,
        ],
        compiler_params=pltpu.CompilerParams(use_tc_tiling_on_sc=False),
    )
    def k(idx_hbm, table_hbm, out_hbm, idx_v, rows_v, sem):
        wid = lax.axis_index("s") * _NC + lax.axis_index("c")
        row_base = wid * rows_per_w

        def body(i, carry):
            row0 = row_base + i * _ROWS_PER_IT
            pltpu.sync_copy(idx_hbm.at[pl.ds(row0, _ROWS_PER_IT)], idx_v)
            copies = []
            for j in range(_ROWS_PER_IT):
                copies.append(
                    pltpu.async_copy(
                        table_hbm.at[idx_v.at[j]],
                        rows_v.at[pl.ds(j * _IDX_W, _IDX_W)],
                        sem,
                    )
                )
            for c in copies:
                c.wait()
            pltpu.sync_copy(rows_v, out_hbm.at[pl.ds(row0 * _IDX_W, _CHUNK)])
            return carry

        lax.fori_loop(0, iters, body, 0)

    return k


def kernel(location, embedding_matrix):
    b, t = location.shape
    dim = embedding_matrix.shape[1]
    n_total = b * t
    idx2d = location.reshape(n_total // _IDX_W, _IDX_W).astype(jnp.int32)
    out = _make_sc_gather(idx2d.shape[0], dim)(idx2d, embedding_matrix)
    return out.reshape(b, t, dim)


# speedup vs baseline: 4.9643x; 1.0449x over previous
import functools

import jax
import jax.numpy as jnp
from jax import lax
from jax.experimental import pallas as pl
from jax.experimental.pallas import tpu as pltpu
from jax.experimental.pallas import tpu_sc as plsc

_INFO = plsc.get_sparse_core_info()
_NC, _NS = _INFO.num_cores, _INFO.num_subcores
_NW = _NC * _NS

_IDX_W = 128
_R = 2
_CHUNK = _R * _IDX_W
_NBUF = 4


def _make_sc_gather(n_rows128, dim):
    assert n_rows128 % (_NW * _R * _NBUF) == 0
    rows_per_w = n_rows128 // _NW
    n_chunks = rows_per_w // _R
    n_blocks = n_chunks // _NBUF
    assert n_blocks >= 3
    n_total = n_rows128 * _IDX_W

    mesh = plsc.VectorSubcoreMesh(core_axis_name="c", subcore_axis_name="s")

    @functools.partial(
        pl.kernel,
        mesh=mesh,
        out_type=jax.ShapeDtypeStruct((n_total, dim), jnp.float32),
        scratch_types=(
            [pltpu.VMEM((_R, _IDX_W), jnp.int32) for _ in range(_NBUF)]
            + [pltpu.VMEM((_CHUNK, dim), jnp.float32) for _ in range(_NBUF)]
            + [pltpu.SemaphoreType.DMA] * (3 * _NBUF)
        ),
        compiler_params=pltpu.CompilerParams(use_tc_tiling_on_sc=False),
    )
    def k(idx_hbm, table_hbm, out_hbm, *scr):
        idx_v = scr[:_NBUF]
        rows_v = scr[_NBUF:2 * _NBUF]
        sem_i = scr[2 * _NBUF:3 * _NBUF]
        sem_g = scr[3 * _NBUF:4 * _NBUF]
        sem_o = scr[4 * _NBUF:5 * _NBUF]

        wid = lax.axis_index("s") * _NC + lax.axis_index("c")
        row_base = wid * rows_per_w

        def idx_load(c, b):
            pltpu.async_copy(
                idx_hbm.at[pl.ds(row_base + c * _R, _R)], idx_v[b], sem_i[b])

        def idx_wait(b):
            pltpu.make_async_copy(
                idx_hbm.at[pl.ds(0, _R)], idx_v[b], sem_i[b]).wait()

        def gathers(b):
            for j in range(_R):
                pltpu.async_copy(
                    table_hbm.at[idx_v[b].at[j]],
                    rows_v[b].at[pl.ds(j * _IDX_W, _IDX_W)],
                    sem_g[b],
                )

        def gathers_wait(b):
            pltpu.make_async_copy(
                table_hbm.at[pl.ds(0, _CHUNK)], rows_v[b], sem_g[b]).wait()

        def out_write(c, b):
            pltpu.async_copy(
                rows_v[b], out_hbm.at[pl.ds((row_base + c * _R) * _IDX_W, _CHUNK)],
                sem_o[b])

        def out_wait(b):
            pltpu.make_async_copy(
                rows_v[b], out_hbm.at[pl.ds(0, _CHUNK)], sem_o[b]).wait()

        for b in range(_NBUF - 1):
            idx_load(b, b)
        for s in range(_NBUF):
            if s >= 1:
                gathers_wait((s - 1) % _NBUF)
                out_write(s - 1, (s - 1) % _NBUF)
            idx_wait(s % _NBUF)
            gathers(s % _NBUF)
            idx_load(s + _NBUF - 1, (s + _NBUF - 1) % _NBUF)

        def block(i2, carry):
            for b in range(_NBUF):
                s = i2 * _NBUF + b
                gathers_wait((b - 1) % _NBUF)
                out_write(s - 1, (b - 1) % _NBUF)
                idx_wait(b)
                out_wait(b)
                gathers(b)
                idx_load(s + _NBUF - 1, (b - 1) % _NBUF)
            return carry

        lax.fori_loop(1, n_blocks - 1, block, 0)

        for b in range(_NBUF):
            s = (n_blocks - 1) * _NBUF + b
            gathers_wait((b - 1) % _NBUF)
            out_write(s - 1, (b - 1) % _NBUF)
            idx_wait(b)
            out_wait(b)
            gathers(b)
            if s + _NBUF - 1 <= n_chunks - 1:
                idx_load(s + _NBUF - 1, (b - 1) % _NBUF)
        gathers_wait(_NBUF - 1)
        out_write(n_chunks - 1, _NBUF - 1)
        for b in range(_NBUF):
            out_wait(b)

    return k


def kernel(location, embedding_matrix):
    b, t = location.shape
    dim = embedding_matrix.shape[1]
    n_total = b * t
    idx2d = location.reshape(n_total // _IDX_W, _IDX_W).astype(jnp.int32)
    out = _make_sc_gather(idx2d.shape[0], dim)(idx2d, embedding_matrix)
    return out.reshape(b, t, dim)


# speedup vs baseline: 5.1401x; 1.0354x over previous
import functools

import jax
import jax.numpy as jnp
from jax import lax
from jax.experimental import pallas as pl
from jax.experimental.pallas import tpu as pltpu
from jax.experimental.pallas import tpu_sc as plsc

_INFO = plsc.get_sparse_core_info()
_NC, _NS = _INFO.num_cores, _INFO.num_subcores
_NW = _NC * _NS

_NB = 2
_NBUF = 4
_SPLITS = ((0, 104), (104, 96))


def _make_sc_gather(n_b, n_t, dim):
    assert n_b % (_NW * _NB * _NBUF) == 0
    b_per_w = n_b // _NW
    n_chunks = b_per_w // _NB
    n_blocks = n_chunks // _NBUF
    assert n_blocks >= 3

    mesh = plsc.VectorSubcoreMesh(core_axis_name="c", subcore_axis_name="s")

    @functools.partial(
        pl.kernel,
        mesh=mesh,
        out_type=jax.ShapeDtypeStruct((n_b, n_t, dim), jnp.float32),
        scratch_types=(
            [pltpu.VMEM((_NB, n_t), jnp.int32) for _ in range(_NBUF)]
            + [pltpu.VMEM((_NB, n_t, dim), jnp.float32) for _ in range(_NBUF)]
            + [pltpu.SemaphoreType.DMA] * (3 * _NBUF)
        ),
        compiler_params=pltpu.CompilerParams(use_tc_tiling_on_sc=False),
    )
    def k(loc_hbm, table_hbm, out_hbm, *scr):
        idx_v = scr[:_NBUF]
        rows_v = scr[_NBUF:2 * _NBUF]
        sem_i = scr[2 * _NBUF:3 * _NBUF]
        sem_g = scr[3 * _NBUF:4 * _NBUF]
        sem_o = scr[4 * _NBUF:5 * _NBUF]

        wid = lax.axis_index("s") * _NC + lax.axis_index("c")
        b_base = wid * b_per_w

        def idx_load(c, b):
            pltpu.async_copy(
                loc_hbm.at[pl.ds(b_base + c * _NB, _NB)], idx_v[b], sem_i[b])

        def idx_wait(b):
            pltpu.make_async_copy(
                loc_hbm.at[pl.ds(0, _NB)], idx_v[b], sem_i[b]).wait()

        def gathers(b):
            for i in range(_NB):
                for off, ln in _SPLITS:
                    pltpu.async_copy(
                        table_hbm.at[idx_v[b].at[i, pl.ds(off, ln)]],
                        rows_v[b].at[i, pl.ds(off, ln)],
                        sem_g[b],
                    )

        def gathers_wait(b):
            for i in range(_NB):
                pltpu.make_async_copy(
                    table_hbm.at[pl.ds(0, n_t)], rows_v[b].at[i], sem_g[b]).wait()

        def out_write(c, b):
            pltpu.async_copy(
                rows_v[b], out_hbm.at[pl.ds(b_base + c * _NB, _NB)], sem_o[b])

        def out_wait(b):
            pltpu.make_async_copy(
                rows_v[b], out_hbm.at[pl.ds(0, _NB)], sem_o[b]).wait()

        for b in range(_NBUF - 1):
            idx_load(b, b)
        for s in range(_NBUF):
            if s >= 1:
                gathers_wait((s - 1) % _NBUF)
                out_write(s - 1, (s - 1) % _NBUF)
            idx_wait(s % _NBUF)
            gathers(s % _NBUF)
            idx_load(s + _NBUF - 1, (s + _NBUF - 1) % _NBUF)

        def block(i2, carry):
            for b in range(_NBUF):
                s = i2 * _NBUF + b
                gathers_wait((b - 1) % _NBUF)
                out_write(s - 1, (b - 1) % _NBUF)
                idx_wait(b)
                out_wait(b)
                gathers(b)
                idx_load(s + _NBUF - 1, (b - 1) % _NBUF)
            return carry

        lax.fori_loop(1, n_blocks - 1, block, 0)

        for b in range(_NBUF):
            s = (n_blocks - 1) * _NBUF + b
            gathers_wait((b - 1) % _NBUF)
            out_write(s - 1, (b - 1) % _NBUF)
            idx_wait(b)
            out_wait(b)
            gathers(b)
            if s + _NBUF - 1 <= n_chunks - 1:
                idx_load(s + _NBUF - 1, (b - 1) % _NBUF)
        gathers_wait(_NBUF - 1)
        out_write(n_chunks - 1, _NBUF - 1)
        for b in range(_NBUF):
            out_wait(b)

    return k


def kernel(location, embedding_matrix):
    n_b, n_t = location.shape
    dim = embedding_matrix.shape[1]
    loc = location.astype(jnp.int32)
    return _make_sc_gather(n_b, n_t, dim)(loc, embedding_matrix)


# speedup vs baseline: 5.5459x; 1.0790x over previous
import functools

import jax
import jax.numpy as jnp
from jax import lax
from jax.experimental import pallas as pl
from jax.experimental.pallas import tpu as pltpu
from jax.experimental.pallas import tpu_sc as plsc

_INFO = plsc.get_sparse_core_info()
_NC, _NS = _INFO.num_cores, _INFO.num_subcores
_NW = _NC * _NS
_L = 16

_SUP = 8
_PAD = 128
_SPLITS = ((0, 128), (128, 72))


def _make_sc_gather(n_b, n_t, dim):
    assert n_b % (_NW * _SUP) == 0 and n_t == 200 and dim == 64
    b_per_w = n_b // _NW
    n_sup = b_per_w // _SUP
    assert n_sup >= 4 and n_sup % 2 == 0

    mesh = plsc.VectorSubcoreMesh(core_axis_name="c", subcore_axis_name="s")

    @functools.partial(
        pl.kernel,
        mesh=mesh,
        out_type=jax.ShapeDtypeStruct((n_b, n_t, dim), jnp.float32),
        scratch_types=(
            [pltpu.VMEM((_SUP * n_t,), jnp.int32) for _ in range(2)]
            + [pltpu.VMEM((n_t, _PAD), jnp.float32) for _ in range(2)]
            + [pltpu.VMEM((1, n_t, dim), jnp.float32) for _ in range(2)]
            + [pltpu.SemaphoreType.DMA] * 6
        ),
    )
    def k(loc_hbm, table_hbm, out_hbm, *scr):
        idx_v = scr[0:2]
        gv = scr[2:4]
        rv = scr[4:6]
        sem_i = scr[6:8]
        sem_g = scr[8:10]
        sem_o = scr[10:12]

        wid = lax.axis_index("s") * _NC + lax.axis_index("c")
        b_base = wid * b_per_w
        n_ch = b_per_w

        def idx_load(sup, si):
            pltpu.async_copy(
                loc_hbm.at[pl.ds((b_base + sup * _SUP) * n_t, _SUP * n_t)],
                idx_v[si], sem_i[si])

        def idx_wait(si):
            pltpu.make_async_copy(
                loc_hbm.at[pl.ds(0, _SUP * n_t)], idx_v[si], sem_i[si]).wait()

        def gathers(r, si, g):
            for off, ln in _SPLITS:
                pltpu.async_copy(
                    table_hbm.at[idx_v[si].at[pl.ds(r * n_t + off, ln)]],
                    gv[g].at[pl.ds(off, ln)],
                    sem_g[g],
                )

        def gathers_wait(g):
            pltpu.make_async_copy(
                table_hbm.at[pl.ds(0, n_t)], gv[g], sem_g[g]).wait()

        def compact(g, rb):
            def body(i8, carry):
                for dr in range(_SUP):
                    r = i8 * _SUP + dr
                    for j in range(dim // _L):
                        rv[rb][0, r, pl.ds(j * _L, _L)] = (
                            gv[g][r, pl.ds(j * _L, _L)])
                return carry
            lax.fori_loop(0, n_t // _SUP, body, 0, unroll=False)

        def out_write(c, rb):
            pltpu.async_copy(
                rv[rb], out_hbm.at[pl.ds(b_base + c, 1)], sem_o[rb])

        def out_wait(rb):
            pltpu.make_async_copy(
                rv[rb], out_hbm.at[pl.ds(0, 1)], sem_o[rb]).wait()

        def step(c, r, si, si_next, last, drain=True):
            g = r % 2
            gathers_wait(g)
            if r == _SUP - 1 and not last:
                idx_wait(si_next)
            if not (last and r == _SUP - 1):
                gathers((r + 1) % _SUP, si if r < _SUP - 1 else si_next,
                        (r + 1) % 2)
            if drain:
                out_wait(g)
            compact(g, g)
            out_write(c, g)

        idx_load(0, 0)
        idx_load(1, 1)
        idx_wait(0)
        gathers(0, 0, 0)
        for r in range(_SUP):
            step(r, r, 0, 1, last=False, drain=(r >= 2))

        def pair_block(s2, carry):
            for ds in range(2):
                s = s2 * 2 + 1 + ds
                si = (1 + ds) % 2
                idx_load(s + 1, (si + 1) % 2)
                for r in range(_SUP):
                    c = s * _SUP + r
                    step(c, r, si, (si + 1) % 2, last=False)
            return carry

        lax.fori_loop(0, (n_sup - 2) // 2, pair_block, 0, unroll=False)

        s = n_sup - 1
        si = s % 2
        for r in range(_SUP):
            c = s * _SUP + r
            step(c, r, si, si, last=True)
        out_wait((n_ch - 2) % 2)
        out_wait((n_ch - 1) % 2)

    return k


def kernel(location, embedding_matrix):
    n_b, n_t = location.shape
    v, dim = embedding_matrix.shape
    loc_flat = location.astype(jnp.int32).reshape(-1)
    table_pad = jnp.pad(embedding_matrix, ((0, 0), (0, _PAD - dim)))
    return _make_sc_gather(n_b, n_t, dim)(loc_flat, table_pad)
